# baseline (speedup 1.0000x reference)
import functools

import jax
import jax.numpy as jnp
from jax import lax
from jax.experimental import pallas as pl
from jax.experimental.pallas import tpu as pltpu
from jax.experimental.pallas import tpu_sc as plsc

_NC, _NS = 2, 16
_NW = _NC * _NS
_E_ROWS = 320000
_W_ROWS = _E_ROWS // 8
_CHUNK = 400
_NCHUNKS = _W_ROWS // _CHUNK
_FULL_ROUNDS = _NCHUNKS // _NW
_TAIL = _NCHUNKS - _FULL_ROUNDS * _NW
_KCHUNK = _CHUNK * 8


def _splat(m16, j):
    return lax.gather(
        m16, jnp.full((16, 1), j, jnp.int32),
        lax.GatherDimensionNumbers(
            offset_dims=(), collapsed_slice_dims=(0,), start_index_map=(0,)),
        (1,), mode=lax.GatherScatterMode.PROMISE_IN_BOUNDS)


def _sc_edge_body(e_hbm, keep_hbm, out_hbm,
                  d0, d1, k0, k1, si0, si1, so0, so1):
    wid = lax.axis_index("s") * _NC + lax.axis_index("c")
    dbuf, kbuf = (d0, d1), (k0, k1)
    sin, sout = (si0, si1), (so0, so1)

    def cid(c):
        return wid + c * _NW if c < _FULL_ROUNDS else _FULL_ROUNDS * _NW + wid

    def rows(c):
        return e_hbm.at[pl.ds(cid(c) * _CHUNK, _CHUNK), :]

    def orows(c):
        return out_hbm.at[pl.ds(cid(c) * _CHUNK, _CHUNK), :]

    def krows(c):
        return keep_hbm.at[pl.ds(cid(c) * _KCHUNK, _KCHUNK)]

    def start_in(c, b):
        pltpu.async_copy(rows(c), dbuf[b], sin[b])
        pltpu.async_copy(krows(c), kbuf[b], sin[b])

    def wait_in(c, b):
        pltpu.make_async_copy(rows(c), dbuf[b], sin[b]).wait()
        pltpu.make_async_copy(krows(c), kbuf[b], sin[b]).wait()

    def compute(b):
        data_v, keep_v = dbuf[b], kbuf[b]

        def group(g, carry):
            m16 = keep_v[pl.ds(g * 16, 16)]
            for j in range(16):
                r = g * 2 + j // 8
                off = (j % 8) * 16
                data_v[r, pl.ds(off, 16)] = (
                    data_v[r, pl.ds(off, 16)] * _splat(m16, j))
            return carry

        lax.fori_loop(0, _KCHUNK // 16, group, 0)

    start_in(0, 0)
    for c in range(_FULL_ROUNDS):
        b = c & 1
        wait_in(c, b)
        compute(b)
        pltpu.async_copy(dbuf[b], orows(c), sout[b])
        if c + 1 < _FULL_ROUNDS:
            if c >= 1:
                pltpu.make_async_copy(dbuf[b ^ 1], orows(c - 1), sout[b ^ 1]).wait()
            start_in(c + 1, b ^ 1)
    last = _FULL_ROUNDS - 1
    pltpu.make_async_copy(dbuf[last & 1], orows(last), sout[last & 1]).wait()
    if _FULL_ROUNDS >= 2:
        pltpu.make_async_copy(dbuf[(last - 1) & 1], orows(last - 1),
                              sout[(last - 1) & 1]).wait()

    @pl.when(wid < _TAIL)
    def _tail():
        pltpu.sync_copy(rows(_FULL_ROUNDS), d0)
        pltpu.sync_copy(krows(_FULL_ROUNDS), k0)
        compute(0)
        pltpu.sync_copy(d0, orows(_FULL_ROUNDS))


_sc_edge = functools.partial(
    pl.kernel,
    out_type=jax.ShapeDtypeStruct((_W_ROWS, 128), jnp.float32),
    mesh=plsc.VectorSubcoreMesh(
        core_axis_name="c", subcore_axis_name="s",
        num_cores=_NC, num_subcores=_NS),
    scratch_types=[
        pltpu.VMEM((_CHUNK, 128), jnp.float32),
        pltpu.VMEM((_CHUNK, 128), jnp.float32),
        pltpu.VMEM((_KCHUNK,), jnp.float32),
        pltpu.VMEM((_KCHUNK,), jnp.float32),
        pltpu.SemaphoreType.DMA,
        pltpu.SemaphoreType.DMA,
        pltpu.SemaphoreType.DMA,
        pltpu.SemaphoreType.DMA,
    ],
)(_sc_edge_body)


def _tc_x_body(x_ref, nm_ref, xo_ref):
    xo_ref[...] = jnp.where(nm_ref[...] != 0, 0.0, x_ref[...])


def kernel(x, edge_attr, node_mask, edge_mask):
    n, d = x.shape
    e, de = edge_attr.shape
    grid = 25
    bn = n // grid
    nm = node_mask.astype(jnp.int32)[:, None]
    x_out = pl.pallas_call(
        _tc_x_body,
        grid=(grid,),
        in_specs=[
            pl.BlockSpec((bn, d), lambda i: (i, 0)),
            pl.BlockSpec((bn, 1), lambda i: (i, 0)),
        ],
        out_specs=pl.BlockSpec((bn, d), lambda i: (i, 0)),
        out_shape=jax.ShapeDtypeStruct((n, d), x.dtype),
    )(x, nm)
    keep = 1.0 - edge_mask.astype(jnp.float32)
    e2 = edge_attr.reshape(_W_ROWS, 128)
    e_out = _sc_edge(e2, keep)
    return (x_out, e_out.reshape(e, de))


# speedup vs baseline: 1.0433x; 1.0433x over previous
import functools

import jax
import jax.numpy as jnp
from jax import lax
from jax.experimental import pallas as pl
from jax.experimental.pallas import tpu as pltpu
from jax.experimental.pallas import tpu_sc as plsc

_NC, _NS = 2, 16
_NW = _NC * _NS
_E_ROWS = 320000
_ROWS_PER_W = _E_ROWS // _NW
_CHUNK = 400
_NCHUNK = _ROWS_PER_W // _CHUNK


def _splat(m16, j):
    return lax.gather(
        m16, jnp.full((16, 1), j, jnp.int32),
        lax.GatherDimensionNumbers(
            offset_dims=(), collapsed_slice_dims=(0,), start_index_map=(0,)),
        (1,), mode=lax.GatherScatterMode.PROMISE_IN_BOUNDS)


_CW = _CHUNK * 16


def _sc_edge_body(e_hbm, keep_hbm, out_hbm,
                  d0, d1, k0, k1, si0, si1, so0, so1):
    wid = lax.axis_index("s") * _NC + lax.axis_index("c")
    base = wid * _ROWS_PER_W
    dbuf, kbuf = (d0, d1), (k0, k1)
    sin, sout = (si0, si1), (so0, so1)
    def words(c):
        return e_hbm.at[pl.ds(base + c * _CHUNK, _CHUNK), :]

    def owords(c):
        return out_hbm.at[pl.ds(base + c * _CHUNK, _CHUNK), :]

    def krows(c):
        return keep_hbm.at[pl.ds(base + c * _CHUNK, _CHUNK)]

    def start_in(c, b):
        pltpu.async_copy(words(c), dbuf[b], sin[b])
        pltpu.async_copy(krows(c), kbuf[b], sin[b])

    def wait_in(c, b):
        pltpu.make_async_copy(words(c), dbuf[b], sin[b]).wait()
        pltpu.make_async_copy(krows(c), kbuf[b], sin[b]).wait()

    def compute(b):
        data_v, keep_v = dbuf[b], kbuf[b]

        def group(g, carry):
            m16 = keep_v[pl.ds(g * 16, 16)]
            for j in range(16):
                r = g * 16 + j
                data_v[r, :] = data_v[r, :] * _splat(m16, j)
            return carry

        lax.fori_loop(0, _CHUNK // 16, group, 0)

    start_in(0, 0)
    for c in range(_NCHUNK):
        b = c & 1
        wait_in(c, b)
        if c + 1 < _NCHUNK:
            if c >= 1:
                pltpu.make_async_copy(dbuf[b ^ 1], owords(c - 1), sout[b ^ 1]).wait()
            start_in(c + 1, b ^ 1)
        compute(b)
        pltpu.async_copy(dbuf[b], owords(c), sout[b])
    pltpu.make_async_copy(dbuf[(_NCHUNK - 1) & 1], owords(_NCHUNK - 1),
                          sout[(_NCHUNK - 1) & 1]).wait()
    if _NCHUNK >= 2:
        pltpu.make_async_copy(dbuf[_NCHUNK & 1], owords(_NCHUNK - 2),
                              sout[_NCHUNK & 1]).wait()


_sc_edge = functools.partial(
    pl.kernel,
    out_type=jax.ShapeDtypeStruct((_E_ROWS, 16), jnp.float32),
    mesh=plsc.VectorSubcoreMesh(
        core_axis_name="c", subcore_axis_name="s",
        num_cores=_NC, num_subcores=_NS),
    scratch_types=[
        pltpu.VMEM((_CHUNK, 16), jnp.float32),
        pltpu.VMEM((_CHUNK, 16), jnp.float32),
        pltpu.VMEM((_CHUNK,), jnp.float32),
        pltpu.VMEM((_CHUNK,), jnp.float32),
        pltpu.SemaphoreType.DMA,
        pltpu.SemaphoreType.DMA,
        pltpu.SemaphoreType.DMA,
        pltpu.SemaphoreType.DMA,
    ],
)(_sc_edge_body)


def _tc_x_body(x_ref, nm_ref, xo_ref):
    xo_ref[...] = jnp.where(nm_ref[...] != 0, 0.0, x_ref[...])


def kernel(x, edge_attr, node_mask, edge_mask):
    n, d = x.shape
    grid = 25
    bn = n // grid
    nm = node_mask.astype(jnp.int32)[:, None]
    x_out = pl.pallas_call(
        _tc_x_body,
        grid=(grid,),
        in_specs=[
            pl.BlockSpec((bn, d), lambda i: (i, 0)),
            pl.BlockSpec((bn, 1), lambda i: (i, 0)),
        ],
        out_specs=pl.BlockSpec((bn, d), lambda i: (i, 0)),
        out_shape=jax.ShapeDtypeStruct((n, d), x.dtype),
    )(x, nm)
    keep = 1.0 - edge_mask.astype(jnp.float32)
    e_out = _sc_edge(edge_attr, keep)
    return (x_out, e_out)
